# Initial kernel scaffold; baseline (speedup 1.0000x reference)
import jax
import jax.numpy as jnp
from jax.experimental import pallas as pl


def kernel(x, edge_index, batch, Wl1, bl1, Wr1, Wl2, bl2, Wr2, Wl3, bl3, Wr3, W1, b1, W2, b2):
    raise NotImplementedError("write your pallas kernel here")


# speedup vs baseline: 1.0352x; 1.0352x over previous
import functools

import jax
import jax.numpy as jnp
from jax.experimental import pallas as pl
from jax.experimental.pallas import tpu as pltpu

N = 10000
G = 64
BM = 1000


def _layer_body(h_ref, s_ref, cnt_ref, wlT_ref, bl_ref, wrT_ref, o_ref, *, relu):
    agg = s_ref[...] / jnp.maximum(cnt_ref[...], 1.0)
    acc = jnp.dot(h_ref[...], wlT_ref[...], preferred_element_type=jnp.float32)
    acc += jnp.dot(agg, wrT_ref[...], preferred_element_type=jnp.float32)
    acc += bl_ref[...]
    if relu:
        acc = jnp.maximum(acc, 0.0)
    o_ref[...] = acc


def _sage_layer(h, s, cnt, Wl, bl, Wr, relu):
    din = h.shape[1]
    dout = Wl.shape[0]
    grid = (N // BM,)
    return pl.pallas_call(
        functools.partial(_layer_body, relu=relu),
        grid=grid,
        in_specs=[
            pl.BlockSpec((BM, din), lambda i: (i, 0)),
            pl.BlockSpec((BM, din), lambda i: (i, 0)),
            pl.BlockSpec((BM, 1), lambda i: (i, 0)),
            pl.BlockSpec((din, dout), lambda i: (0, 0)),
            pl.BlockSpec((1, dout), lambda i: (0, 0)),
            pl.BlockSpec((din, dout), lambda i: (0, 0)),
        ],
        out_specs=pl.BlockSpec((BM, dout), lambda i: (i, 0)),
        out_shape=jax.ShapeDtypeStruct((N, dout), jnp.float32),
    )(h, s, cnt, Wl.T, bl[None, :], Wr.T)


def _poolhead_body(h_ref, b_ref, w1T_ref, b1_ref, w2T_ref, b2_ref, o_ref,
                   ps_ref, pc_ref):
    i = pl.program_id(0)

    @pl.when(i == 0)
    def _init():
        ps_ref[...] = jnp.zeros_like(ps_ref)
        pc_ref[...] = jnp.zeros_like(pc_ref)

    gids = jax.lax.broadcasted_iota(jnp.int32, (BM, G), 1)
    onehot = (b_ref[...] == gids).astype(jnp.float32)
    ps_ref[...] += jnp.dot(onehot.T, h_ref[...], preferred_element_type=jnp.float32)
    pc_ref[...] += jnp.sum(onehot, axis=0, keepdims=True)

    @pl.when(i == pl.num_programs(0) - 1)
    def _head():
        pooled = ps_ref[...] / jnp.maximum(pc_ref[...].T, 1.0)
        z = jnp.dot(pooled, w1T_ref[...], preferred_element_type=jnp.float32)
        z = jnp.maximum(z + b1_ref[...], 0.0)
        o_ref[...] = jnp.dot(z, w2T_ref[...], preferred_element_type=jnp.float32) + b2_ref[...]


def _pool_head(h, batch, W1, b1, W2, b2):
    nhid = W1.shape[1]
    nout = W2.shape[0]
    grid = (N // BM,)
    return pl.pallas_call(
        _poolhead_body,
        grid=grid,
        in_specs=[
            pl.BlockSpec((BM, h.shape[1]), lambda i: (i, 0)),
            pl.BlockSpec((BM, 1), lambda i: (i, 0)),
            pl.BlockSpec((nhid, W1.shape[0]), lambda i: (0, 0)),
            pl.BlockSpec((1, W1.shape[0]), lambda i: (0, 0)),
            pl.BlockSpec((nhid, nout), lambda i: (0, 0)),
            pl.BlockSpec((1, nout), lambda i: (0, 0)),
        ],
        out_specs=pl.BlockSpec((G, nout), lambda i: (0, 0)),
        out_shape=jax.ShapeDtypeStruct((G, nout), jnp.float32),
        scratch_shapes=[
            pltpu.VMEM((G, h.shape[1]), jnp.float32),
            pltpu.VMEM((1, G), jnp.float32),
        ],
    )(h, batch[:, None].astype(jnp.int32), W1.T, b1[None, :], W2.T, b2[None, :])


def _segment_mean_inputs(h, src, dst):
    s = jax.ops.segment_sum(jnp.take(h, src, axis=0), dst, num_segments=N)
    return s


def kernel(x, edge_index, batch, Wl1, bl1, Wr1, Wl2, bl2, Wr2, Wl3, bl3, Wr3, W1, b1, W2, b2):
    src = edge_index[0]
    dst = edge_index[1]
    cnt = jax.ops.segment_sum(jnp.ones((src.shape[0],), jnp.float32), dst,
                              num_segments=N)[:, None]
    h = _sage_layer(x, _segment_mean_inputs(x, src, dst), cnt, Wl1, bl1, Wr1, True)
    h = _sage_layer(h, _segment_mean_inputs(h, src, dst), cnt, Wl2, bl2, Wr2, True)
    h = _sage_layer(h, _segment_mean_inputs(h, src, dst), cnt, Wl3, bl3, Wr3, False)
    return _pool_head(h, batch, W1, b1, W2, b2)


# speedup vs baseline: 2.0642x; 1.9941x over previous
import functools

import jax
import jax.numpy as jnp
from jax import lax
from jax.experimental import pallas as pl
from jax.experimental.pallas import tpu as pltpu
from jax.experimental.pallas import tpu_sc as plsc

N = 10000
E = 160000
G = 64
BM = 1000

NC = 2
NS = 16
EPT = E // NS
EB = 128
NFB = EPT // EB
TAIL = EPT - NFB * EB
NPAD = 10240
HALF = NPAD // 2
TR = 128
ACCR = HALF + 2 * TR
ZR = 128
DEGW = 16


def _agg_body(src_hbm, dst_hbm, h_hbm, agg_hbm,
              srcb, idx1d, dstb, rows_v, zbuf,
              idx16, dstb16, rows16, acc_sh, *, nchunks):
    cid = lax.axis_index("c")
    sid = lax.axis_index("s")
    eoff = sid * EPT

    def fill_z(r, _):
        for g in range(128 // 16):
            zbuf[r, pl.ds(g * 16, 16)] = jnp.zeros((16,), jnp.float32)
        return 0

    lax.fori_loop(0, ZR, fill_z, 0)

    for j in range(nchunks // NC):
        chunk = j * NC + cid
        for half in range(2):
            base = half * HALF
            zoff = sid * (ACCR // NS)
            for z in range(2):
                pltpu.sync_copy(zbuf, acc_sh.at[pl.ds(zoff + z * ZR, ZR)])
            pltpu.sync_copy(zbuf.at[pl.ds(0, ACCR // NS - 2 * ZR)],
                            acc_sh.at[pl.ds(zoff + 2 * ZR, ACCR // NS - 2 * ZR)])
            plsc.subcore_barrier()

            def batch(b, _):
                pltpu.sync_copy(src_hbm.at[pl.ds(eoff + b * EB, EB)], srcb)
                pltpu.sync_copy(dst_hbm.at[pl.ds(eoff + b * EB, EB)], dstb)
                for g in range(EB // 16):
                    dstb[pl.ds(g * 16, 16)] = jnp.minimum(
                        jnp.maximum(dstb[pl.ds(g * 16, 16)] - (base - TR), 0),
                        TR + HALF)
                    idx1d[pl.ds(g * 16, 16)] = (
                        srcb[pl.ds(g * 16, 16)] * nchunks + chunk)
                pltpu.sync_copy(h_hbm.at[idx1d], rows_v)
                pltpu.sync_copy(rows_v, acc_sh.at[dstb], add=True)
                return 0

            lax.fori_loop(0, NFB, batch, 0)

            pltpu.sync_copy(src_hbm.at[pl.ds(eoff + NFB * EB, TAIL)],
                            idx16.at[pl.ds(0, TAIL)])
            pltpu.sync_copy(dst_hbm.at[pl.ds(eoff + NFB * EB, TAIL)], dstb16)
            dstb16[pl.ds(0, 16)] = jnp.minimum(
                jnp.maximum(dstb16[pl.ds(0, 16)] - (base - TR), 0),
                TR + HALF)
            idx16[pl.ds(0, 16)] = idx16[pl.ds(0, 16)] * nchunks + chunk
            pltpu.sync_copy(h_hbm.at[idx16], rows16)
            pltpu.sync_copy(rows16, acc_sh.at[dstb16], add=True)

            plsc.subcore_barrier()

            roff = sid * (HALF // NS)
            pltpu.sync_copy(acc_sh.at[pl.ds(TR + roff, HALF // NS)],
                            agg_hbm.at[chunk, pl.ds(base + roff, HALF // NS)])
            plsc.subcore_barrier()


def _aggregate(h, src, dst, nchunks):
    mesh = plsc.VectorSubcoreMesh(core_axis_name="c", subcore_axis_name="s")
    hk = h.reshape(N * nchunks, 128)
    return pl.kernel(
        functools.partial(_agg_body, nchunks=nchunks),
        out_type=jax.ShapeDtypeStruct((nchunks, NPAD, 128), jnp.float32),
        mesh=mesh,
        scratch_types=[
            pltpu.VMEM((EB,), jnp.int32),
            pltpu.VMEM((EB,), jnp.int32),
            pltpu.VMEM((EB,), jnp.int32),
            pltpu.VMEM((EB, 128), jnp.float32),
            pltpu.VMEM((ZR, 128), jnp.float32),
            pltpu.VMEM((16,), jnp.int32),
            pltpu.VMEM((16,), jnp.int32),
            pltpu.VMEM((16, 128), jnp.float32),
            pltpu.VMEM_SHARED((ACCR, 128), jnp.float32),
        ],
    )(src, dst, hk)


def _layer_body(s_ref, h_ref, cnt_ref, wlT_ref, bl_ref, wrT_ref, o_ref, *, relu):
    k = s_ref.shape[0]
    s = jnp.concatenate([s_ref[i] for i in range(k)], axis=1)
    agg = s / jnp.maximum(cnt_ref[:, 0:1], 1.0)
    acc = jnp.dot(h_ref[...], wlT_ref[...], preferred_element_type=jnp.float32)
    acc += jnp.dot(agg, wrT_ref[...], preferred_element_type=jnp.float32)
    acc += bl_ref[...]
    if relu:
        acc = jnp.maximum(acc, 0.0)
    o_ref[...] = acc


def _sage_layer(h, s, cnt, Wl, bl, Wr, relu):
    din = h.shape[1]
    dout = Wl.shape[0]
    k = s.shape[0]
    grid = (N // BM,)
    return pl.pallas_call(
        functools.partial(_layer_body, relu=relu),
        grid=grid,
        in_specs=[
            pl.BlockSpec((k, BM, 128), lambda i: (0, i, 0)),
            pl.BlockSpec((BM, din), lambda i: (i, 0)),
            pl.BlockSpec((BM, 128), lambda i: (i, 0)),
            pl.BlockSpec((din, dout), lambda i: (0, 0)),
            pl.BlockSpec((1, dout), lambda i: (0, 0)),
            pl.BlockSpec((din, dout), lambda i: (0, 0)),
        ],
        out_specs=pl.BlockSpec((BM, dout), lambda i: (i, 0)),
        out_shape=jax.ShapeDtypeStruct((N, dout), jnp.float32),
    )(s, h, cnt, Wl.T, bl[None, :], Wr.T)


def _poolhead_body(h_ref, b_ref, w1T_ref, b1_ref, w2T_ref, b2_ref, o_ref,
                   ps_ref, pc_ref):
    i = pl.program_id(0)

    @pl.when(i == 0)
    def _init():
        ps_ref[...] = jnp.zeros_like(ps_ref)
        pc_ref[...] = jnp.zeros_like(pc_ref)

    gids = jax.lax.broadcasted_iota(jnp.int32, (BM, G), 1)
    onehot = (b_ref[...] == gids).astype(jnp.float32)
    ps_ref[...] += jnp.dot(onehot.T, h_ref[...], preferred_element_type=jnp.float32)
    pc_ref[...] += jnp.sum(onehot, axis=0, keepdims=True)

    @pl.when(i == pl.num_programs(0) - 1)
    def _head():
        pooled = ps_ref[...] / jnp.maximum(pc_ref[...].T, 1.0)
        z = jnp.dot(pooled, w1T_ref[...], preferred_element_type=jnp.float32)
        z = jnp.maximum(z + b1_ref[...], 0.0)
        o_ref[...] = jnp.dot(z, w2T_ref[...], preferred_element_type=jnp.float32) + b2_ref[...]


def _pool_head(h, batch, W1, b1, W2, b2):
    nhid = W1.shape[1]
    nout = W2.shape[0]
    grid = (N // BM,)
    return pl.pallas_call(
        _poolhead_body,
        grid=grid,
        in_specs=[
            pl.BlockSpec((BM, h.shape[1]), lambda i: (i, 0)),
            pl.BlockSpec((BM, 1), lambda i: (i, 0)),
            pl.BlockSpec((nhid, W1.shape[0]), lambda i: (0, 0)),
            pl.BlockSpec((1, W1.shape[0]), lambda i: (0, 0)),
            pl.BlockSpec((nhid, nout), lambda i: (0, 0)),
            pl.BlockSpec((1, nout), lambda i: (0, 0)),
        ],
        out_specs=pl.BlockSpec((G, nout), lambda i: (0, 0)),
        out_shape=jax.ShapeDtypeStruct((G, nout), jnp.float32),
        scratch_shapes=[
            pltpu.VMEM((G, h.shape[1]), jnp.float32),
            pltpu.VMEM((1, G), jnp.float32),
        ],
    )(h, batch[:, None].astype(jnp.int32), W1.T, b1[None, :], W2.T, b2[None, :])


def _deg_body(dst_hbm, deg_hbm, dstb, dstb16, ones_v, zbuf, dacc_sh):
    cid = lax.axis_index("c")
    sid = lax.axis_index("s")
    eoff = sid * EPT
    base = cid * HALF
    zoff = sid * (ACCR // NS)

    def fill(r, _):
        for g in range(128 // 16):
            zbuf[r, pl.ds(g * 16, 16)] = jnp.zeros((16,), jnp.float32)
            ones_v[r, pl.ds(g * 16, 16)] = jnp.ones((16,), jnp.float32)
        return 0

    lax.fori_loop(0, ZR, fill, 0)
    for z in range(2):
        pltpu.sync_copy(zbuf, dacc_sh.at[pl.ds(zoff + z * ZR, ZR)])
    pltpu.sync_copy(zbuf.at[pl.ds(0, ACCR // NS - 2 * ZR)],
                    dacc_sh.at[pl.ds(zoff + 2 * ZR, ACCR // NS - 2 * ZR)])
    plsc.subcore_barrier()

    def batch(b, _):
        pltpu.sync_copy(dst_hbm.at[pl.ds(eoff + b * EB, EB)], dstb)
        for g in range(EB // 16):
            dstb[pl.ds(g * 16, 16)] = jnp.minimum(
                jnp.maximum(dstb[pl.ds(g * 16, 16)] - (base - TR), 0),
                TR + HALF)
        pltpu.sync_copy(ones_v, dacc_sh.at[dstb], add=True)
        return 0

    lax.fori_loop(0, NFB, batch, 0)
    pltpu.sync_copy(dst_hbm.at[pl.ds(eoff + NFB * EB, TAIL)], dstb16)
    dstb16[pl.ds(0, 16)] = jnp.minimum(
        jnp.maximum(dstb16[pl.ds(0, 16)] - (base - TR), 0), TR + HALF)
    pltpu.sync_copy(ones_v.at[pl.ds(0, TAIL)], dacc_sh.at[dstb16], add=True)
    plsc.subcore_barrier()
    roff = sid * (HALF // NS)
    pltpu.sync_copy(dacc_sh.at[pl.ds(TR + roff, HALF // NS)],
                    deg_hbm.at[pl.ds(base + roff, HALF // NS)])


def _degree(dst):
    mesh = plsc.VectorSubcoreMesh(core_axis_name="c", subcore_axis_name="s")
    return pl.kernel(
        _deg_body,
        out_type=jax.ShapeDtypeStruct((NPAD, 128), jnp.float32),
        mesh=mesh,
        scratch_types=[
            pltpu.VMEM((EB,), jnp.int32),
            pltpu.VMEM((16,), jnp.int32),
            pltpu.VMEM((EB, 128), jnp.float32),
            pltpu.VMEM((ZR, 128), jnp.float32),
            pltpu.VMEM_SHARED((ACCR, 128), jnp.float32),
        ],
    )(dst)


def _t_body(src_hbm, dst_hbm, h_hbm, agg_hbm,
            srcb, idx1d, dstb, rows_v, zbuf, acc_sh, *, mode):
    cid = lax.axis_index("c")
    sid = lax.axis_index("s")
    eoff = sid * EPT
    zoff = sid * (ACCR // NS)

    def fill_z(r, _):
        for g in range(128 // 16):
            zbuf[r, pl.ds(g * 16, 16)] = jnp.zeros((16,), jnp.float32)
        return 0

    lax.fori_loop(0, ZR, fill_z, 0)
    for z in range(2):
        pltpu.sync_copy(zbuf, acc_sh.at[pl.ds(zoff + z * ZR, ZR)])
    pltpu.sync_copy(zbuf.at[pl.ds(0, ACCR // NS - 2 * ZR)],
                    acc_sh.at[pl.ds(zoff + 2 * ZR, ACCR // NS - 2 * ZR)])
    plsc.subcore_barrier()

    def batch(b, _):
        pltpu.sync_copy(src_hbm.at[pl.ds(eoff + b * EB, EB)], srcb)
        pltpu.sync_copy(dst_hbm.at[pl.ds(eoff + b * EB, EB)], dstb)
        for g in range(EB // 16):
            dstb[pl.ds(g * 16, 16)] = jnp.minimum(
                jnp.maximum(dstb[pl.ds(g * 16, 16)] - (0 - TR), 0), TR + HALF)
            idx1d[pl.ds(g * 16, 16)] = srcb[pl.ds(g * 16, 16)] * 2 + cid
        pltpu.sync_copy(h_hbm.at[idx1d], rows_v)
        if mode == 0:
            pltpu.sync_copy(rows_v, acc_sh.at[pl.ds(zoff, EB)])
        elif mode == 1:
            pltpu.sync_copy(rows_v, acc_sh.at[dstb])
        else:
            pltpu.sync_copy(rows_v, acc_sh.at[dstb], add=True)
        return 0

    lax.fori_loop(0, NFB, batch, 0)
    plsc.subcore_barrier()
    roff = sid * (HALF // NS)
    pltpu.sync_copy(acc_sh.at[pl.ds(TR + roff, HALF // NS)],
                    agg_hbm.at[pl.ds(roff, HALF // NS)])
    plsc.subcore_barrier()


def _t_run(h, src, dst, mode):
    mesh = plsc.VectorSubcoreMesh(core_axis_name="c", subcore_axis_name="s")
    hk = h.reshape(N * 2, 128)
    return pl.kernel(
        functools.partial(_t_body, mode=mode),
        out_type=jax.ShapeDtypeStruct((HALF, 128), jnp.float32),
        mesh=mesh,
        scratch_types=[
            pltpu.VMEM((EB,), jnp.int32),
            pltpu.VMEM((EB,), jnp.int32),
            pltpu.VMEM((EB,), jnp.int32),
            pltpu.VMEM((EB, 128), jnp.float32),
            pltpu.VMEM((ZR, 128), jnp.float32),
            pltpu.VMEM_SHARED((ACCR, 128), jnp.float32),
        ],
    )(src, dst, hk)


def kernel(x, edge_index, batch, Wl1, bl1, Wr1, Wl2, bl2, Wr2, Wl3, bl3, Wr3, W1, b1, W2, b2):
    src = edge_index[0]
    dst = edge_index[1]
    cnt = _degree(dst)
    s1 = _aggregate(x, src, dst, 2)
    h = _sage_layer(x, s1, cnt, Wl1, bl1, Wr1, True)
    s2 = _aggregate(h, src, dst, 4)
    h = _sage_layer(h, s2, cnt, Wl2, bl2, Wr2, True)
    s3 = _aggregate(h, src, dst, 4)
    h = _sage_layer(h, s3, cnt, Wl3, bl3, Wr3, False)
    return _pool_head(h, batch, W1, b1, W2, b2)


# speedup vs baseline: 3.0953x; 1.4995x over previous
import functools

import jax
import jax.numpy as jnp
from jax import lax
from jax.experimental import pallas as pl
from jax.experimental.pallas import tpu as pltpu
from jax.experimental.pallas import tpu_sc as plsc

N = 10000
E = 160000
G = 64
BM = 1000

NC = 2
NS = 16
EPT = E // NS
EB = 128
NFB = EPT // EB
TAIL = EPT - NFB * EB
NPAD = 10240
HALF = NPAD // 2
TR = 128
ACCR = HALF + 2 * TR
ZR = 128
DEGW = 16


def _agg_body(src_hbm, dst_hbm, h_hbm, agg_hbm,
              srcb, idx1d, dstb, rows_v, idx2, dstb2, rows_v2, zbuf,
              idx16, dstb16, rows16, gsem0, gsem1, ssem0, ssem1, acc_sh,
              *, nchunks):
    cid = lax.axis_index("c")
    sid = lax.axis_index("s")
    eoff = sid * EPT

    def fill_z(r, _):
        for g in range(128 // 16):
            zbuf[r, pl.ds(g * 16, 16)] = jnp.zeros((16,), jnp.float32)
        return 0

    lax.fori_loop(0, ZR, fill_z, 0)

    for j in range(nchunks // NC):
        chunk = j * NC + cid
        for half in range(2):
            base = half * HALF
            zoff = sid * (ACCR // NS)
            for z in range(2):
                pltpu.sync_copy(zbuf, acc_sh.at[pl.ds(zoff + z * ZR, ZR)])
            pltpu.sync_copy(zbuf.at[pl.ds(0, ACCR // NS - 2 * ZR)],
                            acc_sh.at[pl.ds(zoff + 2 * ZR, ACCR // NS - 2 * ZR)])
            plsc.subcore_barrier()

            def build(b, srcv, dstv, idxv):
                pltpu.sync_copy(src_hbm.at[pl.ds(eoff + b * EB, EB)], srcv)
                pltpu.sync_copy(dst_hbm.at[pl.ds(eoff + b * EB, EB)], dstv)
                for g in range(EB // 16):
                    dstv[pl.ds(g * 16, 16)] = jnp.minimum(
                        jnp.maximum(dstv[pl.ds(g * 16, 16)] - (base - TR), 0),
                        TR + HALF)
                    idxv[pl.ds(g * 16, 16)] = (
                        srcv[pl.ds(g * 16, 16)] * nchunks + chunk)

            build(0, srcb, dstb, idx1d)
            pltpu.async_copy(h_hbm.at[idx1d], rows_v, gsem0)
            build(1, srcb, dstb2, idx2)
            pltpu.async_copy(h_hbm.at[idx2], rows_v2, gsem1)
            pltpu.make_async_copy(h_hbm.at[idx1d], rows_v, gsem0).wait()
            pltpu.async_copy(rows_v, acc_sh.at[dstb], ssem0, add=True)
            pltpu.make_async_copy(h_hbm.at[idx2], rows_v2, gsem1).wait()
            pltpu.async_copy(rows_v2, acc_sh.at[dstb2], ssem1, add=True)

            def pipe(i, _):
                b = 2 * i
                pltpu.make_async_copy(rows_v, acc_sh.at[dstb], ssem0).wait()
                build(b, srcb, dstb, idx1d)
                pltpu.async_copy(h_hbm.at[idx1d], rows_v, gsem0)
                pltpu.make_async_copy(rows_v2, acc_sh.at[dstb2], ssem1).wait()
                build(b + 1, srcb, dstb2, idx2)
                pltpu.async_copy(h_hbm.at[idx2], rows_v2, gsem1)
                pltpu.make_async_copy(h_hbm.at[idx1d], rows_v, gsem0).wait()
                pltpu.async_copy(rows_v, acc_sh.at[dstb], ssem0, add=True)
                pltpu.make_async_copy(h_hbm.at[idx2], rows_v2, gsem1).wait()
                pltpu.async_copy(rows_v2, acc_sh.at[dstb2], ssem1, add=True)
                return 0

            lax.fori_loop(1, NFB // 2, pipe, 0)
            pltpu.make_async_copy(rows_v, acc_sh.at[dstb], ssem0).wait()
            pltpu.make_async_copy(rows_v2, acc_sh.at[dstb2], ssem1).wait()

            pltpu.sync_copy(src_hbm.at[pl.ds(eoff + NFB * EB, TAIL)],
                            idx16.at[pl.ds(0, TAIL)])
            pltpu.sync_copy(dst_hbm.at[pl.ds(eoff + NFB * EB, TAIL)], dstb16)
            dstb16[pl.ds(0, 16)] = jnp.minimum(
                jnp.maximum(dstb16[pl.ds(0, 16)] - (base - TR), 0),
                TR + HALF)
            idx16[pl.ds(0, 16)] = idx16[pl.ds(0, 16)] * nchunks + chunk
            pltpu.sync_copy(h_hbm.at[idx16], rows16)
            pltpu.sync_copy(rows16, acc_sh.at[dstb16], add=True)

            plsc.subcore_barrier()

            roff = sid * (HALF // NS)
            pltpu.sync_copy(acc_sh.at[pl.ds(TR + roff, HALF // NS)],
                            agg_hbm.at[chunk, pl.ds(base + roff, HALF // NS)])
            plsc.subcore_barrier()


def _aggregate(h, src, dst, nchunks):
    mesh = plsc.VectorSubcoreMesh(core_axis_name="c", subcore_axis_name="s")
    hk = h.reshape(N * nchunks, 128)
    return pl.kernel(
        functools.partial(_agg_body, nchunks=nchunks),
        out_type=jax.ShapeDtypeStruct((nchunks, NPAD, 128), jnp.float32),
        mesh=mesh,
        scratch_types=[
            pltpu.VMEM((EB,), jnp.int32),
            pltpu.VMEM((EB,), jnp.int32),
            pltpu.VMEM((EB,), jnp.int32),
            pltpu.VMEM((EB, 128), jnp.float32),
            pltpu.VMEM((EB,), jnp.int32),
            pltpu.VMEM((EB,), jnp.int32),
            pltpu.VMEM((EB, 128), jnp.float32),
            pltpu.VMEM((ZR, 128), jnp.float32),
            pltpu.VMEM((16,), jnp.int32),
            pltpu.VMEM((16,), jnp.int32),
            pltpu.VMEM((16, 128), jnp.float32),
            pltpu.SemaphoreType.DMA,
            pltpu.SemaphoreType.DMA,
            pltpu.SemaphoreType.DMA,
            pltpu.SemaphoreType.DMA,
            pltpu.VMEM_SHARED((ACCR, 128), jnp.float32),
        ],
    )(src, dst, hk)


def _layer_body(s_ref, h_ref, cnt_ref, wlT_ref, bl_ref, wrT_ref, o_ref, *, relu):
    k = s_ref.shape[0]
    s = jnp.concatenate([s_ref[i] for i in range(k)], axis=1)
    agg = s / jnp.maximum(cnt_ref[:, 0:1], 1.0)
    acc = jnp.dot(h_ref[...], wlT_ref[...], preferred_element_type=jnp.float32)
    acc += jnp.dot(agg, wrT_ref[...], preferred_element_type=jnp.float32)
    acc += bl_ref[...]
    if relu:
        acc = jnp.maximum(acc, 0.0)
    o_ref[...] = acc


def _sage_layer(h, s, cnt, Wl, bl, Wr, relu):
    din = h.shape[1]
    dout = Wl.shape[0]
    k = s.shape[0]
    grid = (N // BM,)
    return pl.pallas_call(
        functools.partial(_layer_body, relu=relu),
        grid=grid,
        in_specs=[
            pl.BlockSpec((k, BM, 128), lambda i: (0, i, 0)),
            pl.BlockSpec((BM, din), lambda i: (i, 0)),
            pl.BlockSpec((BM, 128), lambda i: (i, 0)),
            pl.BlockSpec((din, dout), lambda i: (0, 0)),
            pl.BlockSpec((1, dout), lambda i: (0, 0)),
            pl.BlockSpec((din, dout), lambda i: (0, 0)),
        ],
        out_specs=pl.BlockSpec((BM, dout), lambda i: (i, 0)),
        out_shape=jax.ShapeDtypeStruct((N, dout), jnp.float32),
    )(s, h, cnt, Wl.T, bl[None, :], Wr.T)


def _poolhead_body(h_ref, b_ref, w1T_ref, b1_ref, w2T_ref, b2_ref, o_ref,
                   ps_ref, pc_ref):
    i = pl.program_id(0)

    @pl.when(i == 0)
    def _init():
        ps_ref[...] = jnp.zeros_like(ps_ref)
        pc_ref[...] = jnp.zeros_like(pc_ref)

    gids = jax.lax.broadcasted_iota(jnp.int32, (BM, G), 1)
    onehot = (b_ref[...] == gids).astype(jnp.float32)
    ps_ref[...] += jnp.dot(onehot.T, h_ref[...], preferred_element_type=jnp.float32)
    pc_ref[...] += jnp.sum(onehot, axis=0, keepdims=True)

    @pl.when(i == pl.num_programs(0) - 1)
    def _head():
        pooled = ps_ref[...] / jnp.maximum(pc_ref[...].T, 1.0)
        z = jnp.dot(pooled, w1T_ref[...], preferred_element_type=jnp.float32)
        z = jnp.maximum(z + b1_ref[...], 0.0)
        o_ref[...] = jnp.dot(z, w2T_ref[...], preferred_element_type=jnp.float32) + b2_ref[...]


def _pool_head(h, batch, W1, b1, W2, b2):
    nhid = W1.shape[1]
    nout = W2.shape[0]
    grid = (N // BM,)
    return pl.pallas_call(
        _poolhead_body,
        grid=grid,
        in_specs=[
            pl.BlockSpec((BM, h.shape[1]), lambda i: (i, 0)),
            pl.BlockSpec((BM, 1), lambda i: (i, 0)),
            pl.BlockSpec((nhid, W1.shape[0]), lambda i: (0, 0)),
            pl.BlockSpec((1, W1.shape[0]), lambda i: (0, 0)),
            pl.BlockSpec((nhid, nout), lambda i: (0, 0)),
            pl.BlockSpec((1, nout), lambda i: (0, 0)),
        ],
        out_specs=pl.BlockSpec((G, nout), lambda i: (0, 0)),
        out_shape=jax.ShapeDtypeStruct((G, nout), jnp.float32),
        scratch_shapes=[
            pltpu.VMEM((G, h.shape[1]), jnp.float32),
            pltpu.VMEM((1, G), jnp.float32),
        ],
    )(h, batch[:, None].astype(jnp.int32), W1.T, b1[None, :], W2.T, b2[None, :])


def _deg_body(dst_hbm, deg_hbm, dstb, dstb16, ones_v, zbuf, dacc_sh):
    cid = lax.axis_index("c")
    sid = lax.axis_index("s")
    eoff = sid * EPT
    base = cid * HALF
    zoff = sid * (ACCR // NS)

    def fill(r, _):
        for g in range(128 // 16):
            zbuf[r, pl.ds(g * 16, 16)] = jnp.zeros((16,), jnp.float32)
            ones_v[r, pl.ds(g * 16, 16)] = jnp.ones((16,), jnp.float32)
        return 0

    lax.fori_loop(0, ZR, fill, 0)
    for z in range(2):
        pltpu.sync_copy(zbuf, dacc_sh.at[pl.ds(zoff + z * ZR, ZR)])
    pltpu.sync_copy(zbuf.at[pl.ds(0, ACCR // NS - 2 * ZR)],
                    dacc_sh.at[pl.ds(zoff + 2 * ZR, ACCR // NS - 2 * ZR)])
    plsc.subcore_barrier()

    def batch(b, _):
        pltpu.sync_copy(dst_hbm.at[pl.ds(eoff + b * EB, EB)], dstb)
        for g in range(EB // 16):
            dstb[pl.ds(g * 16, 16)] = jnp.minimum(
                jnp.maximum(dstb[pl.ds(g * 16, 16)] - (base - TR), 0),
                TR + HALF)
        pltpu.sync_copy(ones_v, dacc_sh.at[dstb], add=True)
        return 0

    lax.fori_loop(0, NFB, batch, 0)
    pltpu.sync_copy(dst_hbm.at[pl.ds(eoff + NFB * EB, TAIL)], dstb16)
    dstb16[pl.ds(0, 16)] = jnp.minimum(
        jnp.maximum(dstb16[pl.ds(0, 16)] - (base - TR), 0), TR + HALF)
    pltpu.sync_copy(ones_v.at[pl.ds(0, TAIL)], dacc_sh.at[dstb16], add=True)
    plsc.subcore_barrier()
    roff = sid * (HALF // NS)
    pltpu.sync_copy(dacc_sh.at[pl.ds(TR + roff, HALF // NS)],
                    deg_hbm.at[pl.ds(base + roff, HALF // NS)])


def _degree(dst):
    mesh = plsc.VectorSubcoreMesh(core_axis_name="c", subcore_axis_name="s")
    return pl.kernel(
        _deg_body,
        out_type=jax.ShapeDtypeStruct((NPAD, 128), jnp.float32),
        mesh=mesh,
        scratch_types=[
            pltpu.VMEM((EB,), jnp.int32),
            pltpu.VMEM((16,), jnp.int32),
            pltpu.VMEM((EB, 128), jnp.float32),
            pltpu.VMEM((ZR, 128), jnp.float32),
            pltpu.VMEM_SHARED((ACCR, 128), jnp.float32),
        ],
    )(dst)


def _t_body(src_hbm, dst_hbm, h_hbm, agg_hbm,
            srcb, idx1d, dstb, rows_v, zbuf, acc_sh, *, mode):
    cid = lax.axis_index("c")
    sid = lax.axis_index("s")
    eoff = sid * EPT
    zoff = sid * (ACCR // NS)

    def fill_z(r, _):
        for g in range(128 // 16):
            zbuf[r, pl.ds(g * 16, 16)] = jnp.zeros((16,), jnp.float32)
        return 0

    lax.fori_loop(0, ZR, fill_z, 0)
    for z in range(2):
        pltpu.sync_copy(zbuf, acc_sh.at[pl.ds(zoff + z * ZR, ZR)])
    pltpu.sync_copy(zbuf.at[pl.ds(0, ACCR // NS - 2 * ZR)],
                    acc_sh.at[pl.ds(zoff + 2 * ZR, ACCR // NS - 2 * ZR)])
    plsc.subcore_barrier()

    def batch(b, _):
        pltpu.sync_copy(src_hbm.at[pl.ds(eoff + b * EB, EB)], srcb)
        pltpu.sync_copy(dst_hbm.at[pl.ds(eoff + b * EB, EB)], dstb)
        for g in range(EB // 16):
            dstb[pl.ds(g * 16, 16)] = jnp.minimum(
                jnp.maximum(dstb[pl.ds(g * 16, 16)] - (0 - TR), 0), TR + HALF)
            idx1d[pl.ds(g * 16, 16)] = srcb[pl.ds(g * 16, 16)] * 2 + cid
        pltpu.sync_copy(h_hbm.at[idx1d], rows_v)
        if mode == 0:
            pltpu.sync_copy(rows_v, acc_sh.at[pl.ds(zoff, EB)])
        elif mode == 1:
            pltpu.sync_copy(rows_v, acc_sh.at[dstb])
        else:
            pltpu.sync_copy(rows_v, acc_sh.at[dstb], add=True)
        return 0

    lax.fori_loop(0, NFB, batch, 0)
    plsc.subcore_barrier()
    roff = sid * (HALF // NS)
    pltpu.sync_copy(acc_sh.at[pl.ds(TR + roff, HALF // NS)],
                    agg_hbm.at[pl.ds(roff, HALF // NS)])
    plsc.subcore_barrier()


def _t_run(h, src, dst, mode):
    mesh = plsc.VectorSubcoreMesh(core_axis_name="c", subcore_axis_name="s")
    hk = h.reshape(N * 2, 128)
    return pl.kernel(
        functools.partial(_t_body, mode=mode),
        out_type=jax.ShapeDtypeStruct((HALF, 128), jnp.float32),
        mesh=mesh,
        scratch_types=[
            pltpu.VMEM((EB,), jnp.int32),
            pltpu.VMEM((EB,), jnp.int32),
            pltpu.VMEM((EB,), jnp.int32),
            pltpu.VMEM((EB, 128), jnp.float32),
            pltpu.VMEM((ZR, 128), jnp.float32),
            pltpu.VMEM_SHARED((ACCR, 128), jnp.float32),
        ],
    )(src, dst, hk)


def kernel(x, edge_index, batch, Wl1, bl1, Wr1, Wl2, bl2, Wr2, Wl3, bl3, Wr3, W1, b1, W2, b2):
    src = edge_index[0]
    dst = edge_index[1]
    cnt = _degree(dst)
    s1 = _aggregate(x, src, dst, 2)
    h = _sage_layer(x, s1, cnt, Wl1, bl1, Wr1, True)
    s2 = _aggregate(h, src, dst, 4)
    h = _sage_layer(h, s2, cnt, Wl2, bl2, Wr2, True)
    s3 = _aggregate(h, src, dst, 4)
    h = _sage_layer(h, s3, cnt, Wl3, bl3, Wr3, False)
    return _pool_head(h, batch, W1, b1, W2, b2)


# speedup vs baseline: 3.0954x; 1.0000x over previous
import functools

import jax
import jax.numpy as jnp
from jax import lax
from jax.experimental import pallas as pl
from jax.experimental.pallas import tpu as pltpu
from jax.experimental.pallas import tpu_sc as plsc

N = 10000
E = 160000
G = 64
BM = 1000

NC = 2
NS = 16
EPT = E // NS
EB = 128
NFB = EPT // EB
TAIL = EPT - NFB * EB
NPAD = 10240
HALF = NPAD // 2
TR = 128
ACCR = HALF + 2 * TR
ZR = 128


def _agg_body(src_hbm, dst_hbm, h_hbm, agg_hbm,
              srcb, idx1d, dstb, rows_v, idx2, dstb2, rows_v2, zbuf,
              idx16, dstb16, rows16, gsem0, gsem1, ssem0, ssem1, acc_sh,
              *, nchunks):
    cid = lax.axis_index("c")
    sid = lax.axis_index("s")
    eoff = sid * EPT

    def fill_z(r, _):
        for g in range(128 // 16):
            zbuf[r, pl.ds(g * 16, 16)] = jnp.zeros((16,), jnp.float32)
        return 0

    lax.fori_loop(0, ZR, fill_z, 0)

    for j in range(nchunks // NC):
        chunk = j * NC + cid
        for half in range(2):
            base = half * HALF
            zoff = sid * (ACCR // NS)
            for z in range(2):
                pltpu.sync_copy(zbuf, acc_sh.at[pl.ds(zoff + z * ZR, ZR)])
            pltpu.sync_copy(zbuf.at[pl.ds(0, ACCR // NS - 2 * ZR)],
                            acc_sh.at[pl.ds(zoff + 2 * ZR, ACCR // NS - 2 * ZR)])
            plsc.subcore_barrier()

            def build(b, srcv, dstv, idxv):
                pltpu.sync_copy(src_hbm.at[pl.ds(eoff + b * EB, EB)], srcv)
                pltpu.sync_copy(dst_hbm.at[pl.ds(eoff + b * EB, EB)], dstv)
                for g in range(EB // 16):
                    dstv[pl.ds(g * 16, 16)] = jnp.minimum(
                        jnp.maximum(dstv[pl.ds(g * 16, 16)] - (base - TR), 0),
                        TR + HALF)
                    idxv[pl.ds(g * 16, 16)] = (
                        srcv[pl.ds(g * 16, 16)] * nchunks + chunk)

            build(0, srcb, dstb, idx1d)
            pltpu.async_copy(h_hbm.at[idx1d], rows_v, gsem0)
            build(1, srcb, dstb2, idx2)
            pltpu.async_copy(h_hbm.at[idx2], rows_v2, gsem1)
            pltpu.make_async_copy(h_hbm.at[idx1d], rows_v, gsem0).wait()
            pltpu.async_copy(rows_v, acc_sh.at[dstb], ssem0, add=True)
            pltpu.make_async_copy(h_hbm.at[idx2], rows_v2, gsem1).wait()
            pltpu.async_copy(rows_v2, acc_sh.at[dstb2], ssem1, add=True)

            def pipe(i, _):
                b = 2 * i
                pltpu.make_async_copy(rows_v, acc_sh.at[dstb], ssem0).wait()
                build(b, srcb, dstb, idx1d)
                pltpu.async_copy(h_hbm.at[idx1d], rows_v, gsem0)
                pltpu.make_async_copy(rows_v2, acc_sh.at[dstb2], ssem1).wait()
                build(b + 1, srcb, dstb2, idx2)
                pltpu.async_copy(h_hbm.at[idx2], rows_v2, gsem1)
                pltpu.make_async_copy(h_hbm.at[idx1d], rows_v, gsem0).wait()
                pltpu.async_copy(rows_v, acc_sh.at[dstb], ssem0, add=True)
                pltpu.make_async_copy(h_hbm.at[idx2], rows_v2, gsem1).wait()
                pltpu.async_copy(rows_v2, acc_sh.at[dstb2], ssem1, add=True)
                return 0

            lax.fori_loop(1, NFB // 2, pipe, 0)
            pltpu.make_async_copy(rows_v, acc_sh.at[dstb], ssem0).wait()
            pltpu.make_async_copy(rows_v2, acc_sh.at[dstb2], ssem1).wait()

            pltpu.sync_copy(src_hbm.at[pl.ds(eoff + NFB * EB, TAIL)],
                            idx16.at[pl.ds(0, TAIL)])
            pltpu.sync_copy(dst_hbm.at[pl.ds(eoff + NFB * EB, TAIL)], dstb16)
            dstb16[pl.ds(0, 16)] = jnp.minimum(
                jnp.maximum(dstb16[pl.ds(0, 16)] - (base - TR), 0),
                TR + HALF)
            idx16[pl.ds(0, 16)] = idx16[pl.ds(0, 16)] * nchunks + chunk
            pltpu.sync_copy(h_hbm.at[idx16], rows16)
            pltpu.sync_copy(rows16, acc_sh.at[dstb16], add=True)

            plsc.subcore_barrier()

            roff = sid * (HALF // NS)
            pltpu.sync_copy(acc_sh.at[pl.ds(TR + roff, HALF // NS)],
                            agg_hbm.at[chunk, pl.ds(base + roff, HALF // NS)])
            plsc.subcore_barrier()


def _aggregate(h, src, dst, nchunks):
    mesh = plsc.VectorSubcoreMesh(core_axis_name="c", subcore_axis_name="s")
    hk = h.reshape(N * nchunks, 128)
    return pl.kernel(
        functools.partial(_agg_body, nchunks=nchunks),
        out_type=jax.ShapeDtypeStruct((nchunks, NPAD, 128), jnp.float32),
        mesh=mesh,
        scratch_types=[
            pltpu.VMEM((EB,), jnp.int32),
            pltpu.VMEM((EB,), jnp.int32),
            pltpu.VMEM((EB,), jnp.int32),
            pltpu.VMEM((EB, 128), jnp.float32),
            pltpu.VMEM((EB,), jnp.int32),
            pltpu.VMEM((EB,), jnp.int32),
            pltpu.VMEM((EB, 128), jnp.float32),
            pltpu.VMEM((ZR, 128), jnp.float32),
            pltpu.VMEM((16,), jnp.int32),
            pltpu.VMEM((16,), jnp.int32),
            pltpu.VMEM((16, 128), jnp.float32),
            pltpu.SemaphoreType.DMA,
            pltpu.SemaphoreType.DMA,
            pltpu.SemaphoreType.DMA,
            pltpu.SemaphoreType.DMA,
            pltpu.VMEM_SHARED((ACCR, 128), jnp.float32),
        ],
    )(src, dst, hk)


def _layer_body(s_ref, h_ref, cnt_ref, wlT_ref, bl_ref, wrT_ref, o_ref, *, relu):
    k = s_ref.shape[0]
    s = jnp.concatenate([s_ref[i] for i in range(k)], axis=1)
    agg = s / jnp.maximum(cnt_ref[:, 0:1], 1.0)
    acc = jnp.dot(h_ref[...], wlT_ref[...], preferred_element_type=jnp.float32)
    acc += jnp.dot(agg, wrT_ref[...], preferred_element_type=jnp.float32)
    acc += bl_ref[...]
    if relu:
        acc = jnp.maximum(acc, 0.0)
    o_ref[...] = acc


def _sage_layer(h, s, cnt, Wl, bl, Wr, relu):
    din = h.shape[1]
    dout = Wl.shape[0]
    k = s.shape[0]
    grid = (N // BM,)
    return pl.pallas_call(
        functools.partial(_layer_body, relu=relu),
        grid=grid,
        in_specs=[
            pl.BlockSpec((k, BM, 128), lambda i: (0, i, 0)),
            pl.BlockSpec((BM, din), lambda i: (i, 0)),
            pl.BlockSpec((BM, 128), lambda i: (i, 0)),
            pl.BlockSpec((din, dout), lambda i: (0, 0)),
            pl.BlockSpec((1, dout), lambda i: (0, 0)),
            pl.BlockSpec((din, dout), lambda i: (0, 0)),
        ],
        out_specs=pl.BlockSpec((BM, dout), lambda i: (i, 0)),
        out_shape=jax.ShapeDtypeStruct((N, dout), jnp.float32),
    )(s, h, cnt, Wl.T, bl[None, :], Wr.T)


def _poolhead_body(h_ref, b_ref, w1T_ref, b1_ref, w2T_ref, b2_ref, o_ref,
                   ps_ref, pc_ref):
    i = pl.program_id(0)

    @pl.when(i == 0)
    def _init():
        ps_ref[...] = jnp.zeros_like(ps_ref)
        pc_ref[...] = jnp.zeros_like(pc_ref)

    gids = jax.lax.broadcasted_iota(jnp.int32, (BM, G), 1)
    onehot = (b_ref[...] == gids).astype(jnp.float32)
    ps_ref[...] += jnp.dot(onehot.T, h_ref[...], preferred_element_type=jnp.float32)
    pc_ref[...] += jnp.sum(onehot, axis=0, keepdims=True)

    @pl.when(i == pl.num_programs(0) - 1)
    def _head():
        pooled = ps_ref[...] / jnp.maximum(pc_ref[...].T, 1.0)
        z = jnp.dot(pooled, w1T_ref[...], preferred_element_type=jnp.float32)
        z = jnp.maximum(z + b1_ref[...], 0.0)
        o_ref[...] = jnp.dot(z, w2T_ref[...], preferred_element_type=jnp.float32) + b2_ref[...]


def _pool_head(h, batch, W1, b1, W2, b2):
    nhid = W1.shape[1]
    nout = W2.shape[0]
    grid = (N // BM,)
    return pl.pallas_call(
        _poolhead_body,
        grid=grid,
        in_specs=[
            pl.BlockSpec((BM, h.shape[1]), lambda i: (i, 0)),
            pl.BlockSpec((BM, 1), lambda i: (i, 0)),
            pl.BlockSpec((nhid, W1.shape[0]), lambda i: (0, 0)),
            pl.BlockSpec((1, W1.shape[0]), lambda i: (0, 0)),
            pl.BlockSpec((nhid, nout), lambda i: (0, 0)),
            pl.BlockSpec((1, nout), lambda i: (0, 0)),
        ],
        out_specs=pl.BlockSpec((G, nout), lambda i: (0, 0)),
        out_shape=jax.ShapeDtypeStruct((G, nout), jnp.float32),
        scratch_shapes=[
            pltpu.VMEM((G, h.shape[1]), jnp.float32),
            pltpu.VMEM((1, G), jnp.float32),
        ],
    )(h, batch[:, None].astype(jnp.int32), W1.T, b1[None, :], W2.T, b2[None, :])


def _deg_body(dst_hbm, deg_hbm, dstb, dstb16, ones_v, zbuf, dacc_sh):
    cid = lax.axis_index("c")
    sid = lax.axis_index("s")
    eoff = sid * EPT
    base = cid * HALF
    zoff = sid * (ACCR // NS)

    def fill(r, _):
        for g in range(128 // 16):
            zbuf[r, pl.ds(g * 16, 16)] = jnp.zeros((16,), jnp.float32)
            ones_v[r, pl.ds(g * 16, 16)] = jnp.ones((16,), jnp.float32)
        return 0

    lax.fori_loop(0, ZR, fill, 0)
    for z in range(2):
        pltpu.sync_copy(zbuf, dacc_sh.at[pl.ds(zoff + z * ZR, ZR)])
    pltpu.sync_copy(zbuf.at[pl.ds(0, ACCR // NS - 2 * ZR)],
                    dacc_sh.at[pl.ds(zoff + 2 * ZR, ACCR // NS - 2 * ZR)])
    plsc.subcore_barrier()

    def batch(b, _):
        pltpu.sync_copy(dst_hbm.at[pl.ds(eoff + b * EB, EB)], dstb)
        for g in range(EB // 16):
            dstb[pl.ds(g * 16, 16)] = jnp.minimum(
                jnp.maximum(dstb[pl.ds(g * 16, 16)] - (base - TR), 0),
                TR + HALF)
        pltpu.sync_copy(ones_v, dacc_sh.at[dstb], add=True)
        return 0

    lax.fori_loop(0, NFB, batch, 0)
    pltpu.sync_copy(dst_hbm.at[pl.ds(eoff + NFB * EB, TAIL)], dstb16)
    dstb16[pl.ds(0, 16)] = jnp.minimum(
        jnp.maximum(dstb16[pl.ds(0, 16)] - (base - TR), 0), TR + HALF)
    pltpu.sync_copy(ones_v.at[pl.ds(0, TAIL)], dacc_sh.at[dstb16], add=True)
    plsc.subcore_barrier()
    roff = sid * (HALF // NS)
    pltpu.sync_copy(dacc_sh.at[pl.ds(TR + roff, HALF // NS)],
                    deg_hbm.at[pl.ds(base + roff, HALF // NS)])


def _degree(dst):
    mesh = plsc.VectorSubcoreMesh(core_axis_name="c", subcore_axis_name="s")
    return pl.kernel(
        _deg_body,
        out_type=jax.ShapeDtypeStruct((NPAD, 128), jnp.float32),
        mesh=mesh,
        scratch_types=[
            pltpu.VMEM((EB,), jnp.int32),
            pltpu.VMEM((16,), jnp.int32),
            pltpu.VMEM((EB, 128), jnp.float32),
            pltpu.VMEM((ZR, 128), jnp.float32),
            pltpu.VMEM_SHARED((ACCR, 128), jnp.float32),
        ],
    )(dst)


def kernel(x, edge_index, batch, Wl1, bl1, Wr1, Wl2, bl2, Wr2, Wl3, bl3, Wr3, W1, b1, W2, b2):
    src = edge_index[0]
    dst = edge_index[1]
    cnt = _degree(dst)
    s1 = _aggregate(x, src, dst, 2)
    h = _sage_layer(x, s1, cnt, Wl1, bl1, Wr1, True)
    s2 = _aggregate(h, src, dst, 4)
    h = _sage_layer(h, s2, cnt, Wl2, bl2, Wr2, True)
    s3 = _aggregate(h, src, dst, 4)
    h = _sage_layer(h, s3, cnt, Wl3, bl3, Wr3, False)
    return _pool_head(h, batch, W1, b1, W2, b2)
